# baseline (speedup 1.0000x reference)
import functools

import jax
import jax.numpy as jnp
from jax.experimental import pallas as pl
from jax.experimental.pallas import tpu as pltpu

N_NODES = 10000
E_EDGES = 160000
FEAT = 128
WIN = 2048
RB = 256
RBLKS = WIN // RB


def _mlp_body(aD, aT, WD, WT, bs, adj, w1a, w1b, b1, w2, b2, w3, b3,
              fake_ref, out_ref):
    ow = (jnp.dot(aD[...], WD[...], preferred_element_type=jnp.float32)
          + jnp.dot(aT[...], WT[...], preferred_element_type=jnp.float32)
          + bs[...])
    l1 = jnp.clip(jnp.sum(jnp.abs(ow), axis=1, keepdims=True), 1e-12, None)
    fk = ow / l1
    fake_ref[...] = fk
    x1 = jnp.maximum(
        jnp.dot(adj[...], w1a[...], preferred_element_type=jnp.float32)
        + jnp.dot(fk, w1b[...], preferred_element_type=jnp.float32)
        + b1[...], 0.0)
    x2 = jnp.maximum(
        jnp.dot(x1, w2[...], preferred_element_type=jnp.float32) + b2[...],
        0.0)
    out_ref[...] = jax.nn.sigmoid(
        jnp.dot(x2, w3[...], preferred_element_type=jnp.float32) + b3[...])


def _mlp_call(aD, aT, WD, WT, bs, Adj, W1a, W1b, b1, W2, b2, W3, b3):
    full = lambda shape: pl.BlockSpec(shape, lambda j: (0,) * len(shape))
    row = lambda shape: pl.BlockSpec(shape, lambda j: (j, 0))
    return pl.pallas_call(
        _mlp_body,
        grid=(RBLKS,),
        in_specs=[
            row((RB, FEAT)), row((RB, FEAT)),
            full((FEAT, FEAT)), full((FEAT, FEAT)),
            full((1, FEAT)),
            row((RB, N_NODES)),
            full((N_NODES, FEAT)),
            full((FEAT, FEAT)),
            full((1, FEAT)),
            full((FEAT, 64)),
            full((1, 64)),
            full((64, N_NODES)),
            full((1, N_NODES)),
        ],
        out_specs=[
            row((RB, FEAT)),
            row((RB, N_NODES)),
        ],
        out_shape=[
            jax.ShapeDtypeStruct((WIN, FEAT), jnp.float32),
            jax.ShapeDtypeStruct((WIN, N_NODES), jnp.float32),
        ],
    )(aD, aT, WD, WT, bs, Adj, W1a, W1b, b1, W2, b2, W3, b3)


def _conv_window(x_src, eidx, left):
    src = eidx[0]
    dst = eidx[1]
    deg_out = jax.ops.segment_sum(jnp.ones((E_EDGES,), jnp.float32), src,
                                  num_segments=N_NODES)
    ns = jax.lax.rsqrt(jnp.clip(deg_out, 1.0, None))
    g = x_src * ns[:, None]
    inwin = (dst >= left) & (dst < left + WIN)
    dstw = jnp.where(inwin, dst - left, WIN)
    agg = jax.ops.segment_sum(jnp.take(g, src, axis=0), dstw,
                              num_segments=WIN + 1)[:WIN]
    hist = jax.ops.segment_sum(jnp.ones((E_EDGES,), jnp.float32), dstw,
                               num_segments=WIN + 1)[:WIN]
    nd = jax.lax.rsqrt(jnp.clip(hist, 1.0, None))
    return agg * nd[:, None]


def kernel(h_mirna, h_disease, h_target, eidx_MvsD, eidx_DvsM, eidx_MvsT,
           eidx_TvsM, eidx_TvsD, eidx_DvsT, W_MvsD, b_MvsD, W_DvsM, b_DvsM,
           W_MvsT, b_MvsT, W_TvsM, b_TvsM, W_TvsD, b_TvsD, W_DvsT, b_DvsT,
           Adj, W1, b1, W2, b2, W3, b3, size, leftIndex):
    left = jnp.asarray(leftIndex, jnp.int32)
    aD = _conv_window(h_disease, eidx_DvsM, left)
    aT = _conv_window(h_target, eidx_TvsM, left)
    bsum = (b_DvsM + b_TvsM).reshape(1, FEAT)
    W1a = W1[:N_NODES]
    W1b = W1[N_NODES:]
    fake, out = _mlp_call(aD, aT, W_DvsM, W_TvsM, bsum, Adj, W1a, W1b,
                          b1.reshape(1, -1), W2, b2.reshape(1, -1), W3,
                          b3.reshape(1, -1))
    return (fake, out)


# speedup vs baseline: 6.7266x; 6.7266x over previous
import functools

import jax
import jax.numpy as jnp
from jax import lax
from jax.experimental import pallas as pl
from jax.experimental.pallas import tpu as pltpu
from jax.experimental.pallas import tpu_sc as plsc

N_NODES = 10000
E_EDGES = 160000
FEAT = 128
WIN = 2048
RB = 256
RBLKS = WIN // RB

EPT = E_EDGES // 16
NVEC = EPT // 16
CBUF = EPT + 368
CCHUNK = 128
AGG_ROWS = 2176
HISTW = 2064


def _sc_mesh():
    return plsc.VectorSubcoreMesh(core_axis_name="c", subcore_axis_name="s")


def _deg_body(src_hbm, out_hbm, src_v, hist_v):
    c = lax.axis_index("c")
    s = lax.axis_index("s")
    pltpu.sync_copy(src_hbm.at[pl.ds(c * E_EDGES + s * EPT, EPT)], src_v)
    zeros = jnp.zeros((16,), jnp.float32)
    ones = jnp.ones((16,), jnp.float32)

    def zbody(i, _):
        hist_v[pl.ds(i * 16, 16)] = zeros
        return 0

    lax.fori_loop(0, N_NODES // 16, zbody, 0, unroll=8)

    def body(i, _):
        idx = src_v[pl.ds(i * 16, 16)]
        plsc.addupdate_scatter(hist_v, [idx], ones)
        return 0

    lax.fori_loop(0, NVEC, body, 0, unroll=4)
    pltpu.sync_copy(hist_v, out_hbm.at[pl.ds((c * 16 + s) * N_NODES, N_NODES)])


def _deg_call(src_cat):
    f = pl.kernel(
        _deg_body,
        mesh=_sc_mesh(),
        compiler_params=pltpu.CompilerParams(needs_layout_passes=False),
        out_type=jax.ShapeDtypeStruct((2 * 16 * N_NODES,), jnp.float32),
        scratch_types=[
            pltpu.VMEM((EPT,), jnp.int32),
            pltpu.VMEM((N_NODES,), jnp.float32),
        ],
    )
    return f(src_cat)


def _agg_body(edges_hbm, g_hbm, leftv_hbm, agg_out, hist_out,
              src_v, dst_v, csrc, cdw, rows_v, histw, leftv_v,
              agg_sp, sem):
    c = lax.axis_index("c")
    s = lax.axis_index("s")
    pltpu.sync_copy(edges_hbm.at[pl.ds((c * 2 + 0) * E_EDGES + s * EPT, EPT)],
                    src_v)
    pltpu.sync_copy(edges_hbm.at[pl.ds((c * 2 + 1) * E_EDGES + s * EPT, EPT)],
                    dst_v)
    pltpu.sync_copy(leftv_hbm, leftv_v)
    left16 = leftv_v[...]

    zf = jnp.zeros((16,), jnp.float32)
    onesf = jnp.ones((16,), jnp.float32)

    def zr(i, _):
        rows_v[i >> 3, pl.ds((i & 7) * 16, 16)] = zf
        return 0

    lax.fori_loop(0, CCHUNK * FEAT // 16, zr, 0, unroll=8)

    def zh(i, _):
        histw[pl.ds(i * 16, 16)] = zf
        return 0

    lax.fori_loop(0, HISTW // 16, zh, 0, unroll=8)

    pltpu.sync_copy(rows_v.at[pl.ds(0, 128)], agg_sp.at[pl.ds(s * 136, 128)])
    pltpu.sync_copy(rows_v.at[pl.ds(0, 8)], agg_sp.at[pl.ds(s * 136 + 128, 8)])
    plsc.subcore_barrier()

    coff = lax.broadcast(c * N_NODES, (16,))

    def cbody(i, off):
        s16 = src_v[pl.ds(i * 16, 16)]
        d16 = dst_v[pl.ds(i * 16, 16)]
        dw = d16 - left16
        m = (dw >= 0) & (dw < WIN)
        dwc = jnp.where(m, dw, WIN)
        plsc.addupdate_scatter(histw, [dwc], onesf, mask=m)
        plsc.store_compressed(csrc.at[pl.ds(off, 16)], s16 + coff, mask=m)
        plsc.store_compressed(cdw.at[pl.ds(off, 16)], dwc, mask=m)
        return off + jnp.sum(m.astype(jnp.int32))

    off = lax.fori_loop(0, NVEC, cbody, jnp.int32(0))

    z16i = jnp.zeros((16,), jnp.int32)
    dummy16 = jnp.full((16,), WIN, jnp.int32)
    for t in range(CCHUNK // 16):
        csrc[pl.ds(off + t * 16, 16)] = z16i
        cdw[pl.ds(off + t * 16, 16)] = dummy16

    nchunks = (off + CCHUNK - 1) // CCHUNK

    def gbody(k, _):
        idxs = csrc.at[pl.ds(k * CCHUNK, CCHUNK)]
        pltpu.async_copy(g_hbm.at[idxs], rows_v, sem).wait()
        for j in range(CCHUNK // 16):
            dwv = cdw[pl.ds(k * CCHUNK + j * 16, 16)]
            pltpu.sync_copy(rows_v.at[pl.ds(j * 16, 16)], agg_sp.at[dwv],
                            add=True)
        return 0

    lax.fori_loop(0, nchunks, gbody, 0)
    plsc.subcore_barrier()

    pltpu.sync_copy(agg_sp.at[pl.ds(s * 128, 128)],
                    agg_out.at[pl.ds(c * WIN + s * 128, 128)])
    pltpu.sync_copy(histw, hist_out.at[pl.ds((c * 16 + s) * HISTW, HISTW)])


def _agg_call(edges_cat, g_all, leftv):
    f = pl.kernel(
        _agg_body,
        mesh=_sc_mesh(),
        compiler_params=pltpu.CompilerParams(needs_layout_passes=False),
        out_type=[
            jax.ShapeDtypeStruct((2 * WIN, FEAT), jnp.float32),
            jax.ShapeDtypeStruct((2 * 16 * HISTW,), jnp.float32),
        ],
        scratch_types=[
            pltpu.VMEM((EPT,), jnp.int32),
            pltpu.VMEM((EPT,), jnp.int32),
            pltpu.VMEM((CBUF,), jnp.int32),
            pltpu.VMEM((CBUF,), jnp.int32),
            pltpu.VMEM((CCHUNK, FEAT), jnp.float32),
            pltpu.VMEM((HISTW,), jnp.float32),
            pltpu.VMEM((16,), jnp.int32),
            pltpu.VMEM_SHARED((AGG_ROWS, FEAT), jnp.float32),
            pltpu.SemaphoreType.DMA,
        ],
    )
    return f(edges_cat, g_all, leftv)


def _mlp_body(aD, aT, WD, WT, bs, adj, w1a, w1b, b1, w2, b2, w3, b3,
              fake_ref, out_ref):
    ow = (jnp.dot(aD[...], WD[...], preferred_element_type=jnp.float32)
          + jnp.dot(aT[...], WT[...], preferred_element_type=jnp.float32)
          + bs[...])
    l1 = jnp.clip(jnp.sum(jnp.abs(ow), axis=1, keepdims=True), 1e-12, None)
    fk = ow / l1
    fake_ref[...] = fk
    x1 = jnp.maximum(
        jnp.dot(adj[...], w1a[...], preferred_element_type=jnp.float32)
        + jnp.dot(fk, w1b[...], preferred_element_type=jnp.float32)
        + b1[...], 0.0)
    x2 = jnp.maximum(
        jnp.dot(x1, w2[...], preferred_element_type=jnp.float32) + b2[...],
        0.0)
    out_ref[...] = jax.nn.sigmoid(
        jnp.dot(x2, w3[...], preferred_element_type=jnp.float32) + b3[...])


def _mlp_call(aD, aT, WD, WT, bs, Adj, W1a, W1b, b1, W2, b2, W3, b3):
    full = lambda shape: pl.BlockSpec(shape, lambda j: (0,) * len(shape))
    row = lambda shape: pl.BlockSpec(shape, lambda j: (j, 0))
    return pl.pallas_call(
        _mlp_body,
        grid=(RBLKS,),
        in_specs=[
            row((RB, FEAT)), row((RB, FEAT)),
            full((FEAT, FEAT)), full((FEAT, FEAT)),
            full((1, FEAT)),
            row((RB, N_NODES)),
            full((N_NODES, FEAT)),
            full((FEAT, FEAT)),
            full((1, FEAT)),
            full((FEAT, 64)),
            full((1, 64)),
            full((64, N_NODES)),
            full((1, N_NODES)),
        ],
        out_specs=[
            row((RB, FEAT)),
            row((RB, N_NODES)),
        ],
        out_shape=[
            jax.ShapeDtypeStruct((WIN, FEAT), jnp.float32),
            jax.ShapeDtypeStruct((WIN, N_NODES), jnp.float32),
        ],
    )(aD, aT, WD, WT, bs, Adj, W1a, W1b, b1, W2, b2, W3, b3)


def kernel(h_mirna, h_disease, h_target, eidx_MvsD, eidx_DvsM, eidx_MvsT,
           eidx_TvsM, eidx_TvsD, eidx_DvsT, W_MvsD, b_MvsD, W_DvsM, b_DvsM,
           W_MvsT, b_MvsT, W_TvsM, b_TvsM, W_TvsD, b_TvsD, W_DvsT, b_DvsT,
           Adj, W1, b1, W2, b2, W3, b3, size, leftIndex):
    left = jnp.asarray(leftIndex, jnp.int32)

    src_cat = jnp.concatenate([eidx_DvsM[0], eidx_TvsM[0]])
    deg = _deg_call(src_cat).reshape(2, 16, N_NODES).sum(axis=1)
    ns = lax.rsqrt(jnp.clip(deg, 1.0, None))
    g_all = jnp.concatenate([h_disease * ns[0][:, None],
                             h_target * ns[1][:, None]], axis=0)

    edges_cat = jnp.stack([eidx_DvsM, eidx_TvsM]).reshape(-1)
    leftv = jnp.full((16,), left, jnp.int32)
    agg, histp = _agg_call(edges_cat, g_all, leftv)
    agg = agg.reshape(2, WIN, FEAT)
    hist = histp.reshape(2, 16, HISTW).sum(axis=1)[:, :WIN]
    nd = lax.rsqrt(jnp.clip(hist, 1.0, None))
    aD = agg[0] * nd[0][:, None]
    aT = agg[1] * nd[1][:, None]
    bsum = (b_DvsM + b_TvsM).reshape(1, FEAT)
    W1a = W1[:N_NODES]
    W1b = W1[N_NODES:]
    fake, out = _mlp_call(aD, aT, W_DvsM, W_TvsM, bsum, Adj, W1a, W1b,
                          b1.reshape(1, -1), W2, b2.reshape(1, -1), W3,
                          b3.reshape(1, -1))
    return (fake, out)


# speedup vs baseline: 6.7469x; 1.0030x over previous
import functools

import jax
import jax.numpy as jnp
from jax import lax
from jax.experimental import pallas as pl
from jax.experimental.pallas import tpu as pltpu
from jax.experimental.pallas import tpu_sc as plsc

N_NODES = 10000
E_EDGES = 160000
FEAT = 128
WIN = 2048
RB = 256
RBLKS = WIN // RB

EPT = E_EDGES // 16
NVEC = EPT // 16
CBUF = EPT + 368
CCHUNK = 128
AGG_ROWS = 2176
HISTW = 2064


def _sc_mesh():
    return plsc.VectorSubcoreMesh(core_axis_name="c", subcore_axis_name="s")


def _deg_body(src_hbm, out_hbm, src_v, hist_v):
    c = lax.axis_index("c")
    s = lax.axis_index("s")
    pltpu.sync_copy(src_hbm.at[pl.ds(c * E_EDGES + s * EPT, EPT)], src_v)
    zeros = jnp.zeros((16,), jnp.float32)
    ones = jnp.ones((16,), jnp.float32)

    def zbody(i, _):
        hist_v[pl.ds(i * 16, 16)] = zeros
        return 0

    lax.fori_loop(0, N_NODES // 16, zbody, 0, unroll=8)

    def body(i, _):
        idx = src_v[pl.ds(i * 16, 16)]
        plsc.addupdate_scatter(hist_v, [idx], ones)
        return 0

    lax.fori_loop(0, NVEC, body, 0, unroll=4)
    pltpu.sync_copy(hist_v, out_hbm.at[pl.ds((c * 16 + s) * N_NODES, N_NODES)])


def _deg_call(src_cat):
    f = pl.kernel(
        _deg_body,
        mesh=_sc_mesh(),
        compiler_params=pltpu.CompilerParams(needs_layout_passes=False),
        out_type=jax.ShapeDtypeStruct((2 * 16 * N_NODES,), jnp.float32),
        scratch_types=[
            pltpu.VMEM((EPT,), jnp.int32),
            pltpu.VMEM((N_NODES,), jnp.float32),
        ],
    )
    return f(src_cat)


def _agg_body(edges_hbm, g_hbm, leftv_hbm, agg_out, hist_out,
              src_v, dst_v, csrc, cdw, rows_a, rows_b, histw, leftv_v,
              agg_sp, sem_ga, sem_gb, sem_sa, sem_sb):
    c = lax.axis_index("c")
    s = lax.axis_index("s")
    pltpu.sync_copy(edges_hbm.at[pl.ds((c * 2 + 0) * E_EDGES + s * EPT, EPT)],
                    src_v)
    pltpu.sync_copy(edges_hbm.at[pl.ds((c * 2 + 1) * E_EDGES + s * EPT, EPT)],
                    dst_v)
    pltpu.sync_copy(leftv_hbm, leftv_v)
    left16 = leftv_v[...]

    zf = jnp.zeros((16,), jnp.float32)
    onesf = jnp.ones((16,), jnp.float32)

    def zr(i, _):
        rows_a[i >> 3, pl.ds((i & 7) * 16, 16)] = zf
        return 0

    lax.fori_loop(0, CCHUNK * FEAT // 16, zr, 0, unroll=8)

    def zh(i, _):
        histw[pl.ds(i * 16, 16)] = zf
        return 0

    lax.fori_loop(0, HISTW // 16, zh, 0, unroll=8)

    pltpu.sync_copy(rows_a.at[pl.ds(0, 128)], agg_sp.at[pl.ds(s * 136, 128)])
    pltpu.sync_copy(rows_a.at[pl.ds(0, 8)], agg_sp.at[pl.ds(s * 136 + 128, 8)])
    plsc.subcore_barrier()

    coff = lax.broadcast(c * N_NODES, (16,))

    def cbody(i, off):
        s16 = src_v[pl.ds(i * 16, 16)]
        d16 = dst_v[pl.ds(i * 16, 16)]
        dw = d16 - left16
        m = (dw >= 0) & (dw < WIN)
        dwc = jnp.where(m, dw, WIN)
        plsc.addupdate_scatter(histw, [dwc], onesf, mask=m)
        plsc.store_compressed(csrc.at[pl.ds(off, 16)], s16 + coff, mask=m)
        plsc.store_compressed(cdw.at[pl.ds(off, 16)], dwc, mask=m)
        return off + jnp.sum(m.astype(jnp.int32))

    off = lax.fori_loop(0, NVEC, cbody, jnp.int32(0))

    z16i = jnp.zeros((16,), jnp.int32)
    dummy16 = jnp.full((16,), WIN, jnp.int32)
    for t in range(CCHUNK // 16):
        csrc[pl.ds(off + t * 16, 16)] = z16i
        cdw[pl.ds(off + t * 16, 16)] = dummy16

    nchunks = (off + CCHUNK - 1) // CCHUNK

    def issue_gather(k, buf, sem):
        pltpu.async_copy(g_hbm.at[csrc.at[pl.ds(k * CCHUNK, CCHUNK)]],
                         buf, sem)

    def wait_gather(buf, sem):
        pltpu.make_async_copy(g_hbm.at[pl.ds(0, CCHUNK)], buf, sem).wait()

    def issue_scatters(k, buf, sem):
        for j in range(CCHUNK // 16):
            dwv = cdw[pl.ds(k * CCHUNK + j * 16, 16)]
            pltpu.async_copy(buf.at[pl.ds(j * 16, 16)], agg_sp.at[dwv],
                             sem, add=True)

    def drain_scatters(buf, sem):
        pltpu.make_async_copy(g_hbm.at[pl.ds(0, CCHUNK)], buf, sem).wait()

    @pl.when(nchunks > 0)
    def _():
        issue_gather(0, rows_a, sem_ga)

    def gbody(k, _):
        even = (k & 1) == 0

        @pl.when(even)
        def _():
            wait_gather(rows_a, sem_ga)

            @pl.when(k > 0)
            def _():
                drain_scatters(rows_b, sem_sb)

            @pl.when(k + 1 < nchunks)
            def _():
                issue_gather(k + 1, rows_b, sem_gb)

            issue_scatters(k, rows_a, sem_sa)

        @pl.when(jnp.logical_not(even))
        def _():
            wait_gather(rows_b, sem_gb)
            drain_scatters(rows_a, sem_sa)

            @pl.when(k + 1 < nchunks)
            def _():
                issue_gather(k + 1, rows_a, sem_ga)

            issue_scatters(k, rows_b, sem_sb)

        return 0

    lax.fori_loop(0, nchunks, gbody, 0)

    le = (nchunks & 1) == 1

    @pl.when((nchunks >= 1) & le)
    def _():
        drain_scatters(rows_a, sem_sa)

    @pl.when((nchunks >= 1) & jnp.logical_not(le))
    def _():
        drain_scatters(rows_b, sem_sb)

    plsc.subcore_barrier()

    pltpu.sync_copy(agg_sp.at[pl.ds(s * 128, 128)],
                    agg_out.at[pl.ds(c * WIN + s * 128, 128)])
    pltpu.sync_copy(histw, hist_out.at[pl.ds((c * 16 + s) * HISTW, HISTW)])


def _agg_call(edges_cat, g_all, leftv):
    f = pl.kernel(
        _agg_body,
        mesh=_sc_mesh(),
        compiler_params=pltpu.CompilerParams(needs_layout_passes=False),
        out_type=[
            jax.ShapeDtypeStruct((2 * WIN, FEAT), jnp.float32),
            jax.ShapeDtypeStruct((2 * 16 * HISTW,), jnp.float32),
        ],
        scratch_types=[
            pltpu.VMEM((EPT,), jnp.int32),
            pltpu.VMEM((EPT,), jnp.int32),
            pltpu.VMEM((CBUF,), jnp.int32),
            pltpu.VMEM((CBUF,), jnp.int32),
            pltpu.VMEM((CCHUNK, FEAT), jnp.float32),
            pltpu.VMEM((CCHUNK, FEAT), jnp.float32),
            pltpu.VMEM((HISTW,), jnp.float32),
            pltpu.VMEM((16,), jnp.int32),
            pltpu.VMEM_SHARED((AGG_ROWS, FEAT), jnp.float32),
            pltpu.SemaphoreType.DMA,
            pltpu.SemaphoreType.DMA,
            pltpu.SemaphoreType.DMA,
            pltpu.SemaphoreType.DMA,
        ],
    )
    return f(edges_cat, g_all, leftv)


def _mlp_body(aD, aT, WD, WT, bs, adj, w1a, w1b, b1, w2, b2, w3, b3,
              fake_ref, out_ref):
    ow = (jnp.dot(aD[...], WD[...], preferred_element_type=jnp.float32)
          + jnp.dot(aT[...], WT[...], preferred_element_type=jnp.float32)
          + bs[...])
    l1 = jnp.clip(jnp.sum(jnp.abs(ow), axis=1, keepdims=True), 1e-12, None)
    fk = ow / l1
    fake_ref[...] = fk
    x1 = jnp.maximum(
        jnp.dot(adj[...], w1a[...], preferred_element_type=jnp.float32)
        + jnp.dot(fk, w1b[...], preferred_element_type=jnp.float32)
        + b1[...], 0.0)
    x2 = jnp.maximum(
        jnp.dot(x1, w2[...], preferred_element_type=jnp.float32) + b2[...],
        0.0)
    out_ref[...] = jax.nn.sigmoid(
        jnp.dot(x2, w3[...], preferred_element_type=jnp.float32) + b3[...])


def _mlp_call(aD, aT, WD, WT, bs, Adj, W1a, W1b, b1, W2, b2, W3, b3):
    full = lambda shape: pl.BlockSpec(shape, lambda j: (0,) * len(shape))
    row = lambda shape: pl.BlockSpec(shape, lambda j: (j, 0))
    return pl.pallas_call(
        _mlp_body,
        grid=(RBLKS,),
        in_specs=[
            row((RB, FEAT)), row((RB, FEAT)),
            full((FEAT, FEAT)), full((FEAT, FEAT)),
            full((1, FEAT)),
            row((RB, N_NODES)),
            full((N_NODES, FEAT)),
            full((FEAT, FEAT)),
            full((1, FEAT)),
            full((FEAT, 64)),
            full((1, 64)),
            full((64, N_NODES)),
            full((1, N_NODES)),
        ],
        out_specs=[
            row((RB, FEAT)),
            row((RB, N_NODES)),
        ],
        out_shape=[
            jax.ShapeDtypeStruct((WIN, FEAT), jnp.float32),
            jax.ShapeDtypeStruct((WIN, N_NODES), jnp.float32),
        ],
    )(aD, aT, WD, WT, bs, Adj, W1a, W1b, b1, W2, b2, W3, b3)


def kernel(h_mirna, h_disease, h_target, eidx_MvsD, eidx_DvsM, eidx_MvsT,
           eidx_TvsM, eidx_TvsD, eidx_DvsT, W_MvsD, b_MvsD, W_DvsM, b_DvsM,
           W_MvsT, b_MvsT, W_TvsM, b_TvsM, W_TvsD, b_TvsD, W_DvsT, b_DvsT,
           Adj, W1, b1, W2, b2, W3, b3, size, leftIndex):
    left = jnp.asarray(leftIndex, jnp.int32)

    src_cat = jnp.concatenate([eidx_DvsM[0], eidx_TvsM[0]])
    deg = _deg_call(src_cat).reshape(2, 16, N_NODES).sum(axis=1)
    ns = lax.rsqrt(jnp.clip(deg, 1.0, None))
    g_all = jnp.concatenate([h_disease * ns[0][:, None],
                             h_target * ns[1][:, None]], axis=0)

    edges_cat = jnp.stack([eidx_DvsM, eidx_TvsM]).reshape(-1)
    leftv = jnp.full((16,), left, jnp.int32)
    agg, histp = _agg_call(edges_cat, g_all, leftv)
    agg = agg.reshape(2, WIN, FEAT)
    hist = histp.reshape(2, 16, HISTW).sum(axis=1)[:, :WIN]
    nd = lax.rsqrt(jnp.clip(hist, 1.0, None))
    aD = agg[0] * nd[0][:, None]
    aT = agg[1] * nd[1][:, None]
    bsum = (b_DvsM + b_TvsM).reshape(1, FEAT)
    W1a = W1[:N_NODES]
    W1b = W1[N_NODES:]
    fake, out = _mlp_call(aD, aT, W_DvsM, W_TvsM, bsum, Adj, W1a, W1b,
                          b1.reshape(1, -1), W2, b2.reshape(1, -1), W3,
                          b3.reshape(1, -1))
    return (fake, out)


# speedup vs baseline: 6.7567x; 1.0015x over previous
import functools

import jax
import jax.numpy as jnp
from jax import lax
from jax.experimental import pallas as pl
from jax.experimental.pallas import tpu as pltpu
from jax.experimental.pallas import tpu_sc as plsc

N_NODES = 10000
E_EDGES = 160000
FEAT = 128
WIN = 2048
RB = 256
RBLKS = WIN // RB

EPT = E_EDGES // 16
NVEC = EPT // 16
CBUF = EPT + 368
CCHUNK = 128
AGG_ROWS = 2176
HISTW = 2064


def _sc_mesh():
    return plsc.VectorSubcoreMesh(core_axis_name="c", subcore_axis_name="s")


def _deg_body(src_hbm, out_hbm, src_v, hist_v):
    c = lax.axis_index("c")
    s = lax.axis_index("s")
    pltpu.sync_copy(src_hbm.at[pl.ds(c * E_EDGES + s * EPT, EPT)], src_v)
    zeros = jnp.zeros((16,), jnp.float32)
    ones = jnp.ones((16,), jnp.float32)

    def zbody(i, _):
        hist_v[pl.ds(i * 16, 16)] = zeros
        return 0

    lax.fori_loop(0, N_NODES // 16, zbody, 0, unroll=8)

    def body(i, _):
        idx = src_v[pl.ds(i * 16, 16)]
        plsc.addupdate_scatter(hist_v, [idx], ones)
        return 0

    lax.fori_loop(0, NVEC, body, 0, unroll=4)
    pltpu.sync_copy(hist_v, out_hbm.at[pl.ds((c * 16 + s) * N_NODES, N_NODES)])


def _deg_call(src_cat):
    f = pl.kernel(
        _deg_body,
        mesh=_sc_mesh(),
        compiler_params=pltpu.CompilerParams(needs_layout_passes=False),
        out_type=jax.ShapeDtypeStruct((2 * 16 * N_NODES,), jnp.float32),
        scratch_types=[
            pltpu.VMEM((EPT,), jnp.int32),
            pltpu.VMEM((N_NODES,), jnp.float32),
        ],
    )
    return f(src_cat)


def _agg_body(edges_hbm, g_hbm, leftv_hbm, agg_out, hist_out,
              src_v, dst_v, csrc, cdw, rows_a, rows_b, histw, leftv_v,
              agg_sp, sem_ga, sem_gb, sem_sa, sem_sb):
    c = lax.axis_index("c")
    s = lax.axis_index("s")
    pltpu.sync_copy(edges_hbm.at[pl.ds((c * 2 + 0) * E_EDGES + s * EPT, EPT)],
                    src_v)
    pltpu.sync_copy(edges_hbm.at[pl.ds((c * 2 + 1) * E_EDGES + s * EPT, EPT)],
                    dst_v)
    pltpu.sync_copy(leftv_hbm, leftv_v)
    left16 = leftv_v[...]

    zf = jnp.zeros((16,), jnp.float32)
    onesf = jnp.ones((16,), jnp.float32)

    def zr(i, _):
        rows_a[i >> 3, pl.ds((i & 7) * 16, 16)] = zf
        return 0

    lax.fori_loop(0, CCHUNK * FEAT // 16, zr, 0, unroll=8)

    def zh(i, _):
        histw[pl.ds(i * 16, 16)] = zf
        return 0

    lax.fori_loop(0, HISTW // 16, zh, 0, unroll=8)

    pltpu.sync_copy(rows_a.at[pl.ds(0, 128)], agg_sp.at[pl.ds(s * 136, 128)])
    pltpu.sync_copy(rows_a.at[pl.ds(0, 8)], agg_sp.at[pl.ds(s * 136 + 128, 8)])
    plsc.subcore_barrier()

    coff = lax.broadcast(c * N_NODES, (16,))

    def cbody(i, off):
        s16 = src_v[pl.ds(i * 16, 16)]
        d16 = dst_v[pl.ds(i * 16, 16)]
        dw = d16 - left16
        m = (dw >= 0) & (dw < WIN)
        dwc = jnp.where(m, dw, WIN)
        plsc.store_compressed(csrc.at[pl.ds(off, 16)], s16 + coff, mask=m)
        plsc.store_compressed(cdw.at[pl.ds(off, 16)], dwc, mask=m)
        return off + jnp.sum(m.astype(jnp.int32))

    off = lax.fori_loop(0, NVEC, cbody, jnp.int32(0), unroll=4)

    z16i = jnp.zeros((16,), jnp.int32)
    dummy16 = jnp.full((16,), WIN, jnp.int32)
    for t in range(CCHUNK // 16):
        csrc[pl.ds(off + t * 16, 16)] = z16i
        cdw[pl.ds(off + t * 16, 16)] = dummy16

    def hbody(i, _):
        plsc.addupdate_scatter(histw, [cdw[pl.ds(i * 16, 16)]], onesf)
        return 0

    lax.fori_loop(0, (off + 15) // 16, hbody, 0)

    nchunks = (off + CCHUNK - 1) // CCHUNK

    def issue_gather(k, buf, sem):
        pltpu.async_copy(g_hbm.at[csrc.at[pl.ds(k * CCHUNK, CCHUNK)]],
                         buf, sem)

    def wait_gather(buf, sem):
        pltpu.make_async_copy(g_hbm.at[pl.ds(0, CCHUNK)], buf, sem).wait()

    def issue_scatters(k, buf, sem):
        for j in range(CCHUNK // 16):
            dwv = cdw[pl.ds(k * CCHUNK + j * 16, 16)]
            pltpu.async_copy(buf.at[pl.ds(j * 16, 16)], agg_sp.at[dwv],
                             sem, add=True)

    def drain_scatters(buf, sem):
        pltpu.make_async_copy(g_hbm.at[pl.ds(0, CCHUNK)], buf, sem).wait()

    @pl.when(nchunks > 0)
    def _():
        issue_gather(0, rows_a, sem_ga)

    def gbody(k, _):
        even = (k & 1) == 0

        @pl.when(even)
        def _():
            wait_gather(rows_a, sem_ga)

            @pl.when(k > 0)
            def _():
                drain_scatters(rows_b, sem_sb)

            @pl.when(k + 1 < nchunks)
            def _():
                issue_gather(k + 1, rows_b, sem_gb)

            issue_scatters(k, rows_a, sem_sa)

        @pl.when(jnp.logical_not(even))
        def _():
            wait_gather(rows_b, sem_gb)
            drain_scatters(rows_a, sem_sa)

            @pl.when(k + 1 < nchunks)
            def _():
                issue_gather(k + 1, rows_a, sem_ga)

            issue_scatters(k, rows_b, sem_sb)

        return 0

    lax.fori_loop(0, nchunks, gbody, 0)

    le = (nchunks & 1) == 1

    @pl.when((nchunks >= 1) & le)
    def _():
        drain_scatters(rows_a, sem_sa)

    @pl.when((nchunks >= 1) & jnp.logical_not(le))
    def _():
        drain_scatters(rows_b, sem_sb)

    plsc.subcore_barrier()

    pltpu.sync_copy(agg_sp.at[pl.ds(s * 128, 128)],
                    agg_out.at[pl.ds(c * WIN + s * 128, 128)])
    pltpu.sync_copy(histw, hist_out.at[pl.ds((c * 16 + s) * HISTW, HISTW)])


def _agg_call(edges_cat, g_all, leftv):
    f = pl.kernel(
        _agg_body,
        mesh=_sc_mesh(),
        compiler_params=pltpu.CompilerParams(needs_layout_passes=False),
        out_type=[
            jax.ShapeDtypeStruct((2 * WIN, FEAT), jnp.float32),
            jax.ShapeDtypeStruct((2 * 16 * HISTW,), jnp.float32),
        ],
        scratch_types=[
            pltpu.VMEM((EPT,), jnp.int32),
            pltpu.VMEM((EPT,), jnp.int32),
            pltpu.VMEM((CBUF,), jnp.int32),
            pltpu.VMEM((CBUF,), jnp.int32),
            pltpu.VMEM((CCHUNK, FEAT), jnp.float32),
            pltpu.VMEM((CCHUNK, FEAT), jnp.float32),
            pltpu.VMEM((HISTW,), jnp.float32),
            pltpu.VMEM((16,), jnp.int32),
            pltpu.VMEM_SHARED((AGG_ROWS, FEAT), jnp.float32),
            pltpu.SemaphoreType.DMA,
            pltpu.SemaphoreType.DMA,
            pltpu.SemaphoreType.DMA,
            pltpu.SemaphoreType.DMA,
        ],
    )
    return f(edges_cat, g_all, leftv)


def _mlp_body(aD, aT, WD, WT, bs, adj, w1a, w1b, b1, w2, b2, w3, b3,
              fake_ref, out_ref):
    ow = (jnp.dot(aD[...], WD[...], preferred_element_type=jnp.float32)
          + jnp.dot(aT[...], WT[...], preferred_element_type=jnp.float32)
          + bs[...])
    l1 = jnp.clip(jnp.sum(jnp.abs(ow), axis=1, keepdims=True), 1e-12, None)
    fk = ow / l1
    fake_ref[...] = fk
    x1 = jnp.maximum(
        jnp.dot(adj[...], w1a[...], preferred_element_type=jnp.float32)
        + jnp.dot(fk, w1b[...], preferred_element_type=jnp.float32)
        + b1[...], 0.0)
    x2 = jnp.maximum(
        jnp.dot(x1, w2[...], preferred_element_type=jnp.float32) + b2[...],
        0.0)
    out_ref[...] = jax.nn.sigmoid(
        jnp.dot(x2, w3[...], preferred_element_type=jnp.float32) + b3[...])


def _mlp_call(aD, aT, WD, WT, bs, Adj, W1a, W1b, b1, W2, b2, W3, b3):
    full = lambda shape: pl.BlockSpec(shape, lambda j: (0,) * len(shape))
    row = lambda shape: pl.BlockSpec(shape, lambda j: (j, 0))
    return pl.pallas_call(
        _mlp_body,
        grid=(RBLKS,),
        in_specs=[
            row((RB, FEAT)), row((RB, FEAT)),
            full((FEAT, FEAT)), full((FEAT, FEAT)),
            full((1, FEAT)),
            row((RB, N_NODES)),
            full((N_NODES, FEAT)),
            full((FEAT, FEAT)),
            full((1, FEAT)),
            full((FEAT, 64)),
            full((1, 64)),
            full((64, N_NODES)),
            full((1, N_NODES)),
        ],
        out_specs=[
            row((RB, FEAT)),
            row((RB, N_NODES)),
        ],
        out_shape=[
            jax.ShapeDtypeStruct((WIN, FEAT), jnp.float32),
            jax.ShapeDtypeStruct((WIN, N_NODES), jnp.float32),
        ],
    )(aD, aT, WD, WT, bs, Adj, W1a, W1b, b1, W2, b2, W3, b3)


def kernel(h_mirna, h_disease, h_target, eidx_MvsD, eidx_DvsM, eidx_MvsT,
           eidx_TvsM, eidx_TvsD, eidx_DvsT, W_MvsD, b_MvsD, W_DvsM, b_DvsM,
           W_MvsT, b_MvsT, W_TvsM, b_TvsM, W_TvsD, b_TvsD, W_DvsT, b_DvsT,
           Adj, W1, b1, W2, b2, W3, b3, size, leftIndex):
    left = jnp.asarray(leftIndex, jnp.int32)

    src_cat = jnp.concatenate([eidx_DvsM[0], eidx_TvsM[0]])
    deg = _deg_call(src_cat).reshape(2, 16, N_NODES).sum(axis=1)
    ns = lax.rsqrt(jnp.clip(deg, 1.0, None))
    g_all = jnp.concatenate([h_disease * ns[0][:, None],
                             h_target * ns[1][:, None]], axis=0)

    edges_cat = jnp.stack([eidx_DvsM, eidx_TvsM]).reshape(-1)
    leftv = jnp.full((16,), left, jnp.int32)
    agg, histp = _agg_call(edges_cat, g_all, leftv)
    agg = agg.reshape(2, WIN, FEAT)
    hist = histp.reshape(2, 16, HISTW).sum(axis=1)[:, :WIN]
    nd = lax.rsqrt(jnp.clip(hist, 1.0, None))
    aD = agg[0] * nd[0][:, None]
    aT = agg[1] * nd[1][:, None]
    bsum = (b_DvsM + b_TvsM).reshape(1, FEAT)
    W1a = W1[:N_NODES]
    W1b = W1[N_NODES:]
    fake, out = _mlp_call(aD, aT, W_DvsM, W_TvsM, bsum, Adj, W1a, W1b,
                          b1.reshape(1, -1), W2, b2.reshape(1, -1), W3,
                          b3.reshape(1, -1))
    return (fake, out)
